# baseline (speedup 1.0000x reference)
import dataclasses
import functools

import jax
import jax.numpy as jnp
from jax import lax
from jax.experimental import pallas as pl
from jax.experimental.pallas import tpu as pltpu
from jax.experimental.pallas import tpu_sc as plsc

GAMMA = 1.0

B = 16384
D = 64
L = 100
LPAD = 112
NBINS = 128

NC, NS = 2, 16
NW = NC * NS
ROWS_W = B // NW
CH = 128


def _sc_compiler_params():
    cp = pltpu.CompilerParams()
    if "needs_layout_passes" in pltpu.CompilerParams.__dataclass_fields__:
        cp = dataclasses.replace(cp, needs_layout_passes=False)
    return cp


def _sc_hist(ids_pad):
    mesh = plsc.VectorSubcoreMesh(core_axis_name="c", subcore_axis_name="s")

    @functools.partial(
        pl.kernel,
        compiler_params=_sc_compiler_params(),
        out_type=jax.ShapeDtypeStruct((B, NBINS), jnp.float32),
        mesh=mesh,
        scratch_types=[
            pltpu.VMEM((CH, LPAD), jnp.int32),
            pltpu.VMEM((CH, NBINS), jnp.float32),
            pltpu.SemaphoreType.DMA,
        ],
    )
    def k(ids_hbm, cnt_out, ids_v, cnt_v, isem):
        wid = lax.axis_index("s") * NC + lax.axis_index("c")
        ones = jnp.ones((16,), jnp.float32)
        zeros16 = jnp.zeros((16,), jnp.float32)
        lastmask = lax.iota(jnp.int32, 16) < (L - (LPAD // 16 - 1) * 16)

        for c in range(ROWS_W // CH):
            base = wid * ROWS_W + c * CH
            icp = pltpu.async_copy(ids_hbm.at[pl.ds(base, CH)], ids_v, isem)

            @pl.loop(0, CH)
            def _(r):
                for kk in range(NBINS // 16):
                    cnt_v[r, pl.ds(kk * 16, 16)] = zeros16

            icp.wait()

            @pl.loop(0, CH)
            def _(r):
                rows = jnp.broadcast_to(r, (16,)).astype(jnp.int32)
                for g in range(LPAD // 16):
                    ids16 = ids_v[r, pl.ds(g * 16, 16)]
                    if g == LPAD // 16 - 1:
                        plsc.addupdate_scatter(
                            cnt_v, [rows, ids16], ones, mask=lastmask)
                    else:
                        plsc.addupdate_scatter(cnt_v, [rows, ids16], ones)

            pltpu.sync_copy(cnt_v, cnt_out.at[pl.ds(base, CH)])

    return k(ids_pad)


def _sc_gather(ent4, heads2d):
    mesh = plsc.VectorSubcoreMesh(core_axis_name="c", subcore_axis_name="s")

    @functools.partial(
        pl.kernel,
        compiler_params=_sc_compiler_params(),
        out_type=jax.ShapeDtypeStruct((B, 128), jnp.int32),
        mesh=mesh,
        scratch_types=[
            pltpu.VMEM((128,), jnp.int32),
            pltpu.VMEM((CH, 128), jnp.int32),
            pltpu.SemaphoreType.DMA,
        ],
    )
    def k(ent_hbm, heads_hbm, hr_out, hidx_v, ebuf_v, gsem):
        wid = lax.axis_index("s") * NC + lax.axis_index("c")

        for c in range(ROWS_W // CH):
            base = wid * ROWS_W + c * CH
            pltpu.sync_copy(heads_hbm.at[base // 128], hidx_v)
            pltpu.async_copy(ent_hbm.at[hidx_v], ebuf_v, gsem).wait()
            pltpu.sync_copy(ebuf_v, hr_out.at[pl.ds(base, CH)])

    return k(ent4, heads2d)


BT = 1024


def _tc_loss_body(cnt_ref, hr_ref, par_ref, rel_ref, cs_ref, rs_ref, out_ref):
    i = pl.program_id(0)
    t_s = jnp.dot(cnt_ref[...], cs_ref[...], preferred_element_type=jnp.float32)
    oh = (rel_ref[...] == lax.broadcasted_iota(jnp.int32, (1, 128), 1)
          ).astype(jnp.float32)
    r_s = jnp.dot(oh, rs_ref[...], preferred_element_type=jnp.float32)
    p0 = par_ref[:, 0:1] > 0.5
    p1 = par_ref[:, 1:2] > 0.5
    w01 = jnp.where(p0, hr_ref[:, 32:64], hr_ref[:, 0:32])
    w23 = jnp.where(p0, hr_ref[:, 96:128], hr_ref[:, 64:96])
    wsel = jnp.where(p1, w23, w01)
    he = lax.bitcast_convert_type(wsel << 16, jnp.float32)
    ho = lax.bitcast_convert_type(
        wsel & jnp.int32(-65536), jnp.float32)
    hs = jnp.concatenate([he, ho], axis=1)
    d = jnp.sum(jnp.abs(hs - t_s + r_s), axis=1)
    p = jnp.sum(jnp.maximum(d + GAMMA, 0.0))

    @pl.when(i == 0)
    def _():
        out_ref[0, 0] = p

    @pl.when(i != 0)
    def _():
        out_ref[0, 0] += p


def _tc_loss(cnt, hr4, par2, relcol, cs, rs):
    return pl.pallas_call(
        _tc_loss_body,
        grid=(B // BT,),
        in_specs=[
            pl.BlockSpec((BT, NBINS), lambda i: (i, 0)),
            pl.BlockSpec((BT, 128), lambda i: (i, 0)),
            pl.BlockSpec((BT, 2), lambda i: (i, 0)),
            pl.BlockSpec((BT, 1), lambda i: (i, 0)),
            pl.BlockSpec((128, D), lambda i: (0, 0)),
            pl.BlockSpec((128, D), lambda i: (0, 0)),
        ],
        out_specs=pl.BlockSpec(memory_space=pltpu.SMEM),
        out_shape=jax.ShapeDtypeStruct((1, 1), jnp.float32),
    )(cnt, hr4, par2, relcol, cs, rs)


def _split_cols(m):
    return jnp.concatenate([m[:, 0::2], m[:, 1::2]], axis=1)


def kernel(entity_embeddings, char_embeddings, rel_attr_embeddings, heads, rels, char_ids):
    heads32 = heads.astype(jnp.int32)
    u = lax.bitcast_convert_type(
        entity_embeddings.astype(jnp.bfloat16), jnp.uint16).astype(jnp.uint32)
    packed = u[:, 0::2] | (u[:, 1::2] << 16)
    ent4 = lax.bitcast_convert_type(packed, jnp.int32).reshape(
        entity_embeddings.shape[0] // 4, 128)
    heads2d = (heads32 // 4).reshape(B // 128, 128)
    q = heads32 % 4
    par2 = jnp.stack([(q % 2).astype(jnp.float32),
                      (q // 2).astype(jnp.float32)], axis=1)
    relcol = rels.astype(jnp.int32).reshape(B, 1)
    ids_pad = jnp.pad(char_ids.astype(jnp.int32), ((0, 0), (0, LPAD - L)))
    cs = _split_cols(char_embeddings)
    rs = jnp.pad(_split_cols(rel_attr_embeddings), ((0, 128 - 22), (0, 0)))
    cnt = _sc_hist(ids_pad)
    hr4 = _sc_gather(ent4, heads2d)
    out = _tc_loss(cnt, hr4, par2, relcol, cs, rs)
    return out[0, 0]


# speedup vs baseline: 3.0161x; 3.0161x over previous
import dataclasses
import functools

import jax
import jax.numpy as jnp
from jax import lax
from jax.experimental import pallas as pl
from jax.experimental.pallas import tpu as pltpu
from jax.experimental.pallas import tpu_sc as plsc

GAMMA = 1.0

B = 16384
D = 64
L = 100
LPAD = 112
NBINS = 128

NC, NS = 2, 16
NW = NC * NS
ROWS_W = B // NW
CH = 128


def _sc_compiler_params():
    cp = pltpu.CompilerParams()
    if "needs_layout_passes" in pltpu.CompilerParams.__dataclass_fields__:
        cp = dataclasses.replace(cp, needs_layout_passes=False)
    return cp


def _sc_hist(ids_pad):
    mesh = plsc.VectorSubcoreMesh(core_axis_name="c", subcore_axis_name="s")

    @functools.partial(
        pl.kernel,
        compiler_params=_sc_compiler_params(),
        out_type=jax.ShapeDtypeStruct((B, NBINS), jnp.float32),
        mesh=mesh,
        scratch_types=[
            pltpu.VMEM((CH, LPAD), jnp.int32),
            pltpu.VMEM((CH, NBINS), jnp.float32),
            pltpu.SemaphoreType.DMA,
        ],
    )
    def k(ids_hbm, cnt_out, ids_v, cnt_v, isem):
        wid = lax.axis_index("s") * NC + lax.axis_index("c")
        ones = jnp.ones((16,), jnp.float32)
        zeros16 = jnp.zeros((16,), jnp.float32)
        lastmask = lax.iota(jnp.int32, 16) < (L - (LPAD // 16 - 1) * 16)

        for c in range(ROWS_W // CH):
            base = wid * ROWS_W + c * CH
            icp = pltpu.async_copy(ids_hbm.at[pl.ds(base, CH)], ids_v, isem)

            @pl.loop(0, CH)
            def _(r):
                for kk in range(NBINS // 16):
                    cnt_v[r, pl.ds(kk * 16, 16)] = zeros16

            icp.wait()

            @pl.loop(0, CH)
            def _(r):
                rows = jnp.broadcast_to(r, (16,)).astype(jnp.int32)
                for g in range(LPAD // 16):
                    ids16 = ids_v[r, pl.ds(g * 16, 16)]
                    if g == LPAD // 16 - 1:
                        plsc.addupdate_scatter(
                            cnt_v, [rows, ids16], ones, mask=lastmask)
                    else:
                        plsc.addupdate_scatter(cnt_v, [rows, ids16], ones)

            pltpu.sync_copy(cnt_v, cnt_out.at[pl.ds(base, CH)])

    return k(ids_pad)


def _sc_gather(ent, heads2d):
    mesh = plsc.VectorSubcoreMesh(core_axis_name="c", subcore_axis_name="s")
    cp = _sc_compiler_params()
    if "use_tc_tiling_on_sc" in pltpu.CompilerParams.__dataclass_fields__:
        cp = dataclasses.replace(cp, use_tc_tiling_on_sc=False)

    @functools.partial(
        pl.kernel,
        compiler_params=cp,
        out_type=jax.ShapeDtypeStruct((B, D), jnp.float32),
        mesh=mesh,
        scratch_types=[
            pltpu.VMEM((128,), jnp.int32),
            pltpu.VMEM((CH, D), jnp.float32),
            pltpu.SemaphoreType.DMA,
        ],
    )
    def k(ent_hbm, heads_hbm, h_out, hidx_v, ebuf_v, gsem):
        wid = lax.axis_index("s") * NC + lax.axis_index("c")

        for c in range(ROWS_W // CH):
            base = wid * ROWS_W + c * CH
            pltpu.sync_copy(heads_hbm.at[base // 128], hidx_v)
            pltpu.async_copy(ent_hbm.at[hidx_v], ebuf_v, gsem).wait()
            pltpu.sync_copy(ebuf_v, h_out.at[pl.ds(base, CH)])

    return k(ent, heads2d)


BT = 1024


def _tc_loss_body(cnt_ref, h_ref, rel_ref, c_ref, rp_ref, out_ref):
    i = pl.program_id(0)
    t = jnp.dot(cnt_ref[...], c_ref[...], preferred_element_type=jnp.float32)
    oh = (rel_ref[...] == lax.broadcasted_iota(jnp.int32, (1, 128), 1)
          ).astype(jnp.float32)
    r = jnp.dot(oh, rp_ref[...], preferred_element_type=jnp.float32)
    d = jnp.sum(jnp.abs(h_ref[...] + r - t), axis=1)
    p = jnp.sum(jnp.maximum(d + GAMMA, 0.0))

    @pl.when(i == 0)
    def _():
        out_ref[0, 0] = p

    @pl.when(i != 0)
    def _():
        out_ref[0, 0] += p


def _tc_loss(cnt, h, relcol, cemb, rp):
    return pl.pallas_call(
        _tc_loss_body,
        grid=(B // BT,),
        in_specs=[
            pl.BlockSpec((BT, NBINS), lambda i: (i, 0)),
            pl.BlockSpec((BT, D), lambda i: (i, 0)),
            pl.BlockSpec((BT, 1), lambda i: (i, 0)),
            pl.BlockSpec((128, D), lambda i: (0, 0)),
            pl.BlockSpec((128, D), lambda i: (0, 0)),
        ],
        out_specs=pl.BlockSpec(memory_space=pltpu.SMEM),
        out_shape=jax.ShapeDtypeStruct((1, 1), jnp.float32),
    )(cnt, h, relcol, cemb, rp)


def kernel(entity_embeddings, char_embeddings, rel_attr_embeddings, heads, rels, char_ids):
    heads2d = heads.astype(jnp.int32).reshape(B // 128, 128)
    relcol = rels.astype(jnp.int32).reshape(B, 1)
    ids_pad = jnp.pad(char_ids.astype(jnp.int32), ((0, 0), (0, LPAD - L)))
    rp = jnp.pad(rel_attr_embeddings, ((0, 128 - 22), (0, 0)))
    cnt = _sc_hist(ids_pad)
    h = _sc_gather(entity_embeddings, heads2d)
    out = _tc_loss(cnt, h, relcol, char_embeddings, rp)
    return out[0, 0]


# speedup vs baseline: 3.1215x; 1.0349x over previous
import dataclasses
import functools

import jax
import jax.numpy as jnp
from jax import lax
from jax.experimental import pallas as pl
from jax.experimental.pallas import tpu as pltpu
from jax.experimental.pallas import tpu_sc as plsc

GAMMA = 1.0

B = 16384
D = 64
L = 100
LPAD = 112
NBINS = 128

NC, NS = 2, 16
NW = NC * NS
ROWS_W = B // NW
CH = 128


def _sc_compiler_params():
    cp = pltpu.CompilerParams()
    if "needs_layout_passes" in pltpu.CompilerParams.__dataclass_fields__:
        cp = dataclasses.replace(cp, needs_layout_passes=False)
    return cp


def _sc_hist(ids_pad):
    mesh = plsc.VectorSubcoreMesh(core_axis_name="c", subcore_axis_name="s")

    @functools.partial(
        pl.kernel,
        compiler_params=_sc_compiler_params(),
        out_type=jax.ShapeDtypeStruct((B, NBINS), jnp.float32),
        mesh=mesh,
        scratch_types=[
            pltpu.VMEM((CH, LPAD), jnp.int32),
            pltpu.VMEM((CH, NBINS), jnp.float32),
            pltpu.SemaphoreType.DMA,
        ],
    )
    def k(ids_hbm, cnt_out, ids_v, cnt_v, isem):
        wid = lax.axis_index("s") * NC + lax.axis_index("c")
        ones = jnp.ones((16,), jnp.float32)
        zeros16 = jnp.zeros((16,), jnp.float32)
        lastmask = lax.iota(jnp.int32, 16) < (L - (LPAD // 16 - 1) * 16)

        for c in range(ROWS_W // CH):
            base = wid * ROWS_W + c * CH
            icp = pltpu.async_copy(ids_hbm.at[pl.ds(base, CH)], ids_v, isem)

            @pl.loop(0, CH)
            def _(r):
                for kk in range(NBINS // 16):
                    cnt_v[r, pl.ds(kk * 16, 16)] = zeros16

            icp.wait()

            @pl.loop(0, CH)
            def _(r):
                rows = jnp.broadcast_to(r, (16,)).astype(jnp.int32)
                for g in range(LPAD // 16):
                    ids16 = ids_v[r, pl.ds(g * 16, 16)]
                    if g == LPAD // 16 - 1:
                        plsc.addupdate_scatter(
                            cnt_v, [rows, ids16], ones, mask=lastmask)
                    else:
                        plsc.addupdate_scatter(cnt_v, [rows, ids16], ones)

            pltpu.sync_copy(cnt_v, cnt_out.at[pl.ds(base, CH)])

    return k(ids_pad)


def _sc_gather(ent2, heads2d):
    mesh = plsc.VectorSubcoreMesh(core_axis_name="c", subcore_axis_name="s")

    @functools.partial(
        pl.kernel,
        compiler_params=_sc_compiler_params(),
        out_type=jax.ShapeDtypeStruct((B, 128), jnp.float32),
        mesh=mesh,
        scratch_types=[
            pltpu.VMEM((128,), jnp.int32),
            pltpu.VMEM((CH, 128), jnp.float32),
            pltpu.SemaphoreType.DMA,
        ],
    )
    def k(ent_hbm, heads_hbm, h_out, hidx_v, ebuf_v, gsem):
        wid = lax.axis_index("s") * NC + lax.axis_index("c")

        for c in range(ROWS_W // CH):
            base = wid * ROWS_W + c * CH
            pltpu.sync_copy(heads_hbm.at[base // 128], hidx_v)
            pltpu.async_copy(ent_hbm.at[hidx_v], ebuf_v, gsem).wait()
            pltpu.sync_copy(ebuf_v, h_out.at[pl.ds(base, CH)])

    return k(ent2, heads2d)


BT = 1024


def _tc_loss_body(cnt_ref, h2_ref, par_ref, rel_ref, c_ref, rp_ref, out_ref):
    i = pl.program_id(0)
    t = jnp.dot(cnt_ref[...], c_ref[...], preferred_element_type=jnp.float32)
    oh = (rel_ref[...] == lax.broadcasted_iota(jnp.int32, (1, 128), 1)
          ).astype(jnp.float32)
    r = jnp.dot(oh, rp_ref[...], preferred_element_type=jnp.float32)
    h = jnp.where(par_ref[...] > 0.5, h2_ref[:, 64:128], h2_ref[:, 0:64])
    d = jnp.sum(jnp.abs(h + r - t), axis=1)
    p = jnp.sum(jnp.maximum(d + GAMMA, 0.0))

    @pl.when(i == 0)
    def _():
        out_ref[0, 0] = p

    @pl.when(i != 0)
    def _():
        out_ref[0, 0] += p


def _tc_loss(cnt, h2, parity, relcol, cemb, rp):
    return pl.pallas_call(
        _tc_loss_body,
        grid=(B // BT,),
        in_specs=[
            pl.BlockSpec((BT, NBINS), lambda i: (i, 0)),
            pl.BlockSpec((BT, 128), lambda i: (i, 0)),
            pl.BlockSpec((BT, 1), lambda i: (i, 0)),
            pl.BlockSpec((BT, 1), lambda i: (i, 0)),
            pl.BlockSpec((128, D), lambda i: (0, 0)),
            pl.BlockSpec((128, D), lambda i: (0, 0)),
        ],
        out_specs=pl.BlockSpec(memory_space=pltpu.SMEM),
        out_shape=jax.ShapeDtypeStruct((1, 1), jnp.float32),
    )(cnt, h2, parity, relcol, cemb, rp)


def kernel(entity_embeddings, char_embeddings, rel_attr_embeddings, heads, rels, char_ids):
    heads32 = heads.astype(jnp.int32)
    ent2 = entity_embeddings.reshape(entity_embeddings.shape[0] // 2, 128)
    heads2d = (heads32 // 2).reshape(B // 128, 128)
    parity = (heads32 % 2).astype(jnp.float32).reshape(B, 1)
    relcol = rels.astype(jnp.int32).reshape(B, 1)
    ids_pad = jnp.pad(char_ids.astype(jnp.int32), ((0, 0), (0, LPAD - L)))
    rp = jnp.pad(rel_attr_embeddings, ((0, 128 - 22), (0, 0)))
    cnt = _sc_hist(ids_pad)
    h2 = _sc_gather(ent2, heads2d)
    out = _tc_loss(cnt, h2, parity, relcol, char_embeddings, rp)
    return out[0, 0]


# speedup vs baseline: 4.6666x; 1.4950x over previous
import dataclasses
import functools

import jax
import jax.numpy as jnp
from jax import lax
from jax.experimental import pallas as pl
from jax.experimental.pallas import tpu as pltpu
from jax.experimental.pallas import tpu_sc as plsc

GAMMA = 1.0

B = 16384
D = 64
L = 100
LPAD = 112
NBINS = 128

NC, NS = 2, 16
NW = NC * NS
ROWS_W = B // NW
CH = 128


def _sc_compiler_params():
    cp = pltpu.CompilerParams()
    if "needs_layout_passes" in pltpu.CompilerParams.__dataclass_fields__:
        cp = dataclasses.replace(cp, needs_layout_passes=False)
    return cp


def _sc_hist(ids_pad):
    mesh = plsc.VectorSubcoreMesh(core_axis_name="c", subcore_axis_name="s")

    @functools.partial(
        pl.kernel,
        compiler_params=_sc_compiler_params(),
        out_type=jax.ShapeDtypeStruct((B, NBINS), jnp.float32),
        mesh=mesh,
        scratch_types=[
            pltpu.VMEM((CH, LPAD), jnp.int32),
            pltpu.VMEM((CH, NBINS), jnp.float32),
            pltpu.SemaphoreType.DMA,
        ],
    )
    def k(ids_hbm, cnt_out, ids_v, cnt_v, isem):
        wid = lax.axis_index("s") * NC + lax.axis_index("c")
        ones = jnp.ones((16,), jnp.float32)
        zeros16 = jnp.zeros((16,), jnp.float32)
        lastmask = lax.iota(jnp.int32, 16) < (L - (LPAD // 16 - 1) * 16)

        for c in range(ROWS_W // CH):
            base = wid * ROWS_W + c * CH
            icp = pltpu.async_copy(ids_hbm.at[pl.ds(base, CH)], ids_v, isem)

            @pl.loop(0, CH)
            def _(r):
                for kk in range(NBINS // 16):
                    cnt_v[r, pl.ds(kk * 16, 16)] = zeros16

            icp.wait()

            @pl.loop(0, CH)
            def _(r):
                rows = jnp.broadcast_to(r, (16,)).astype(jnp.int32)
                for g in range(LPAD // 16):
                    ids16 = ids_v[r, pl.ds(g * 16, 16)]
                    if g == LPAD // 16 - 1:
                        plsc.addupdate_scatter(
                            cnt_v, [rows, ids16], ones, mask=lastmask)
                    else:
                        plsc.addupdate_scatter(cnt_v, [rows, ids16], ones)

            pltpu.sync_copy(cnt_v, cnt_out.at[pl.ds(base, CH)])

    return k(ids_pad)


KF = 16


def _sc_gather_blk(ent, heads1d, cnt):
    mesh = plsc.VectorSubcoreMesh(core_axis_name="c", subcore_axis_name="s")

    @functools.partial(
        pl.kernel,
        compiler_params=_sc_compiler_params(),
        out_type=jax.ShapeDtypeStruct((B, D), jnp.float32),
        mesh=mesh,
        scratch_types=[
            pltpu.VMEM((ROWS_W,), jnp.int32),
            pltpu.VMEM((KF, 8, D), jnp.float32),
            pltpu.VMEM((ROWS_W, D), jnp.float32),
            pltpu.SemaphoreType.DMA,
        ],
    )
    def k(ent_hbm, heads_hbm, cnt_hbm, h_out, hv, tbuf, rbuf, gsem):
        wid = lax.axis_index("s") * NC + lax.axis_index("c")
        base = wid * ROWS_W
        pltpu.sync_copy(heads_hbm.at[pl.ds(base, ROWS_W)], hv)
        iota16 = lax.iota(jnp.int32, 16)

        @pl.loop(0, ROWS_W, step=KF)
        def _(j0):
            hv16 = hv[pl.ds(j0, KF)]
            es = []
            cps = []
            for jj in range(KF):
                e = jnp.max(jnp.where(iota16 == jj, hv16, 0))
                es.append(e)
                e8 = pl.multiple_of((e >> 3) << 3, 8)
                cp = pltpu.make_async_copy(
                    ent_hbm.at[pl.ds(e8, 8)], tbuf.at[jj], gsem)
                cp.start()
                cps.append(cp)
            for cp in cps:
                cp.wait()
            for jj in range(KF):
                r = es[jj] & 7
                for q in range(D // 16):
                    sl = pl.ds(q * 16, 16)
                    rbuf[j0 + jj, sl] = tbuf[jj, r, sl]

        pltpu.sync_copy(rbuf, h_out.at[pl.ds(base, ROWS_W)])

    return k(ent, heads1d, cnt)


BT = 1024


def _tc_loss_body(cnt_ref, h_ref, rel_ref, c_ref, rp_ref, out_ref):
    i = pl.program_id(0)
    t = jnp.dot(cnt_ref[...], c_ref[...], preferred_element_type=jnp.float32)
    oh = (rel_ref[...] == lax.broadcasted_iota(jnp.int32, (1, 128), 1)
          ).astype(jnp.float32)
    r = jnp.dot(oh, rp_ref[...], preferred_element_type=jnp.float32)
    d = jnp.sum(jnp.abs(h_ref[...] + r - t), axis=1)
    p = jnp.sum(jnp.maximum(d + GAMMA, 0.0))

    @pl.when(i == 0)
    def _():
        out_ref[0, 0] = p

    @pl.when(i != 0)
    def _():
        out_ref[0, 0] += p


def _tc_loss(cnt, h, relcol, cemb, rp):
    return pl.pallas_call(
        _tc_loss_body,
        grid=(B // BT,),
        in_specs=[
            pl.BlockSpec((BT, NBINS), lambda i: (i, 0)),
            pl.BlockSpec((BT, D), lambda i: (i, 0)),
            pl.BlockSpec((BT, 1), lambda i: (i, 0)),
            pl.BlockSpec((128, D), lambda i: (0, 0)),
            pl.BlockSpec((128, D), lambda i: (0, 0)),
        ],
        out_specs=pl.BlockSpec(memory_space=pltpu.SMEM),
        out_shape=jax.ShapeDtypeStruct((1, 1), jnp.float32),
    )(cnt, h, relcol, cemb, rp)


def kernel(entity_embeddings, char_embeddings, rel_attr_embeddings, heads, rels, char_ids):
    heads1d = heads.astype(jnp.int32)
    relcol = rels.astype(jnp.int32).reshape(B, 1)
    ids_pad = jnp.pad(char_ids.astype(jnp.int32), ((0, 0), (0, LPAD - L)))
    rp = jnp.pad(rel_attr_embeddings, ((0, 128 - 22), (0, 0)))
    cnt = _sc_hist(ids_pad)
    h = _sc_gather_blk(entity_embeddings, heads1d, cnt)
    out = _tc_loss(cnt, h, relcol, char_embeddings, rp)
    return out[0, 0]


# speedup vs baseline: 4.7684x; 1.0218x over previous
import dataclasses
import functools

import jax
import jax.numpy as jnp
from jax import lax
from jax.experimental import pallas as pl
from jax.experimental.pallas import tpu as pltpu
from jax.experimental.pallas import tpu_sc as plsc

GAMMA = 1.0

B = 16384
D = 64
L = 100
LPAD = 112
NBINS = 128

NC, NS = 2, 16
NW = NC * NS
ROWS_W = B // NW
CH = 128


def _sc_compiler_params():
    cp = pltpu.CompilerParams()
    if "needs_layout_passes" in pltpu.CompilerParams.__dataclass_fields__:
        cp = dataclasses.replace(cp, needs_layout_passes=False)
    return cp


def _sc_hist(ids_pad):
    mesh = plsc.VectorSubcoreMesh(core_axis_name="c", subcore_axis_name="s")

    @functools.partial(
        pl.kernel,
        compiler_params=_sc_compiler_params(),
        out_type=jax.ShapeDtypeStruct((B, NBINS), jnp.float32),
        mesh=mesh,
        scratch_types=[
            pltpu.VMEM((CH, LPAD), jnp.int32),
            pltpu.VMEM((CH, NBINS), jnp.float32),
            pltpu.SemaphoreType.DMA,
        ],
    )
    def k(ids_hbm, cnt_out, ids_v, cnt_v, isem):
        wid = lax.axis_index("s") * NC + lax.axis_index("c")
        ones = jnp.ones((16,), jnp.float32)
        zeros16 = jnp.zeros((16,), jnp.float32)
        lastmask = lax.iota(jnp.int32, 16) < (L - (LPAD // 16 - 1) * 16)

        for c in range(ROWS_W // CH):
            base = wid * ROWS_W + c * CH
            icp = pltpu.async_copy(ids_hbm.at[pl.ds(base, CH)], ids_v, isem)

            @pl.loop(0, CH)
            def _(r):
                for kk in range(NBINS // 16):
                    cnt_v[r, pl.ds(kk * 16, 16)] = zeros16

            icp.wait()

            @pl.loop(0, CH)
            def _(r):
                rows = jnp.broadcast_to(r, (16,)).astype(jnp.int32)
                for g in range(LPAD // 16):
                    ids16 = ids_v[r, pl.ds(g * 16, 16)]
                    if g == LPAD // 16 - 1:
                        plsc.addupdate_scatter(
                            cnt_v, [rows, ids16], ones, mask=lastmask)
                    else:
                        plsc.addupdate_scatter(cnt_v, [rows, ids16], ones)

            pltpu.sync_copy(cnt_v, cnt_out.at[pl.ds(base, CH)])

    return k(ids_pad)


KF = 16


def _sc_gather_blk(ent, heads1d, cnt):
    mesh = plsc.VectorSubcoreMesh(core_axis_name="c", subcore_axis_name="s")

    @functools.partial(
        pl.kernel,
        compiler_params=_sc_compiler_params(),
        out_type=jax.ShapeDtypeStruct((B, D), jnp.float32),
        mesh=mesh,
        scratch_types=[
            pltpu.VMEM((ROWS_W,), jnp.int32),
            pltpu.VMEM((2 * KF, 8, D), jnp.float32),
            pltpu.VMEM((ROWS_W, D), jnp.float32),
            pltpu.SemaphoreType.DMA,
        ],
    )
    def k(ent_hbm, heads_hbm, cnt_hbm, h_out, hv, tbuf, rbuf, gsem):
        wid = lax.axis_index("s") * NC + lax.axis_index("c")
        base = wid * ROWS_W
        pltpu.sync_copy(heads_hbm.at[pl.ds(base, ROWS_W)], hv)
        iota16 = lax.iota(jnp.int32, 16)

        @pl.loop(0, ROWS_W, step=2 * KF)
        def _(j0):
            es = []
            cps = []
            for half in range(2):
                hv16 = hv[pl.ds(j0 + half * KF, KF)]
                for jj in range(KF):
                    e = jnp.max(jnp.where(iota16 == jj, hv16, 0))
                    es.append(e)
                    e8 = pl.multiple_of((e >> 3) << 3, 8)
                    cp = pltpu.make_async_copy(
                        ent_hbm.at[pl.ds(e8, 8)],
                        tbuf.at[half * KF + jj], gsem)
                    cp.start()
                    cps.append(cp)
            for cp in cps:
                cp.wait()
            for sj in range(2 * KF):
                r = es[sj] & 7
                for q in range(D // 16):
                    sl = pl.ds(q * 16, 16)
                    rbuf[j0 + sj, sl] = tbuf[sj, r, sl]

        pltpu.sync_copy(rbuf, h_out.at[pl.ds(base, ROWS_W)])

    return k(ent, heads1d, cnt)


BT = 1024


def _tc_loss_body(cnt_ref, h_ref, rel_ref, c_ref, rp_ref, out_ref):
    i = pl.program_id(0)
    t = jnp.dot(cnt_ref[...], c_ref[...], preferred_element_type=jnp.float32)
    oh = (rel_ref[...] == lax.broadcasted_iota(jnp.int32, (1, 128), 1)
          ).astype(jnp.float32)
    r = jnp.dot(oh, rp_ref[...], preferred_element_type=jnp.float32)
    d = jnp.sum(jnp.abs(h_ref[...] + r - t), axis=1)
    p = jnp.sum(jnp.maximum(d + GAMMA, 0.0))

    @pl.when(i == 0)
    def _():
        out_ref[0, 0] = p

    @pl.when(i != 0)
    def _():
        out_ref[0, 0] += p


def _tc_loss(cnt, h, relcol, cemb, rp):
    return pl.pallas_call(
        _tc_loss_body,
        grid=(B // BT,),
        in_specs=[
            pl.BlockSpec((BT, NBINS), lambda i: (i, 0)),
            pl.BlockSpec((BT, D), lambda i: (i, 0)),
            pl.BlockSpec((BT, 1), lambda i: (i, 0)),
            pl.BlockSpec((128, D), lambda i: (0, 0)),
            pl.BlockSpec((128, D), lambda i: (0, 0)),
        ],
        out_specs=pl.BlockSpec(memory_space=pltpu.SMEM),
        out_shape=jax.ShapeDtypeStruct((1, 1), jnp.float32),
    )(cnt, h, relcol, cemb, rp)


def kernel(entity_embeddings, char_embeddings, rel_attr_embeddings, heads, rels, char_ids):
    heads1d = heads.astype(jnp.int32)
    relcol = rels.astype(jnp.int32).reshape(B, 1)
    ids_pad = jnp.pad(char_ids.astype(jnp.int32), ((0, 0), (0, LPAD - L)))
    rp = jnp.pad(rel_attr_embeddings, ((0, 128 - 22), (0, 0)))
    cnt = _sc_hist(ids_pad)
    h = _sc_gather_blk(entity_embeddings, heads1d, cnt)
    out = _tc_loss(cnt, h, relcol, char_embeddings, rp)
    return out[0, 0]
